# baseline (speedup 1.0000x reference)
import functools
import math

import jax
import jax.numpy as jnp
from jax import lax
from jax.experimental import pallas as pl
from jax.experimental.pallas import tpu as pltpu
from jax.experimental.pallas import tpu_sc as plsc

_D = 64
_SCALE = math.sqrt(_D)
_C = 128


@functools.lru_cache(maxsize=None)
def _make_kernel(B: int):
    info = plsc.get_sparse_core_info()
    nw = info.num_cores * info.num_subcores
    per_w = B // nw
    nch = per_w // _C
    assert per_w % _C == 0

    mesh = plsc.VectorSubcoreMesh(core_axis_name="c", subcore_axis_name="s")

    @functools.partial(
        pl.kernel,
        mesh=mesh,
        out_type=jax.ShapeDtypeStruct((B, _D), jnp.float32),
        compiler_params=pltpu.CompilerParams(use_tc_tiling_on_sc=False),
        scratch_types=[
            pltpu.VMEM((nch, _C), jnp.int32),
            pltpu.VMEM((_C, _D), jnp.float32),
            pltpu.SemaphoreType.DMA,
        ],
    )
    def k(tokens_hbm, table_hbm, out_hbm, idx_v, rows_v, gsem):
        wid = lax.axis_index("s") * info.num_cores + lax.axis_index("c")
        base = wid * per_w
        pltpu.sync_copy(tokens_hbm.at[wid], idx_v)

        def chunk_body(j, carry):
            pltpu.async_copy(table_hbm.at[idx_v.at[j]], rows_v, gsem).wait()

            def scale_body(i, c):
                for t in range(_D // 16):
                    sl = pl.ds(t * 16, 16)
                    rows_v[i, sl] = rows_v[i, sl] * _SCALE
                return c

            lax.fori_loop(0, _C, scale_body, 0)
            pltpu.sync_copy(rows_v, out_hbm.at[pl.ds(base + j * _C, _C)])
            return carry

        lax.fori_loop(0, nch, chunk_body, 0)

    return k


def kernel(tokens, table):
    s0, s1 = tokens.shape
    b = s0 * s1
    flat = tokens.reshape(-1).astype(jnp.int32)
    info = plsc.get_sparse_core_info()
    nw = info.num_cores * info.num_subcores
    grain = nw * _C
    b_pad = (b + grain - 1) // grain * grain
    if b_pad != b:
        flat = jnp.pad(flat, (0, b_pad - b))
    tokens3d = flat.reshape(nw, b_pad // (nw * _C), _C)
    out = _make_kernel(b_pad)(tokens3d, table)
    return out[:b].reshape(s0, s1, _D)


# speedup vs baseline: 1.2140x; 1.2140x over previous
import functools
import math

import jax
import jax.numpy as jnp
from jax import lax
from jax.experimental import pallas as pl
from jax.experimental.pallas import tpu as pltpu
from jax.experimental.pallas import tpu_sc as plsc

_D = 64
_SCALE = math.sqrt(_D)
_C = 128
_NB = 8
_LEAD = 4


@functools.lru_cache(maxsize=None)
def _make_kernel(B: int):
    info = plsc.get_sparse_core_info()
    nw = info.num_cores * info.num_subcores
    per_w = B // nw
    nch = per_w // _C
    assert per_w % _C == 0 and nch % _NB == 0

    mesh = plsc.VectorSubcoreMesh(core_axis_name="c", subcore_axis_name="s")

    scratch = [pltpu.VMEM((nch, _C), jnp.int32)]
    scratch += [pltpu.VMEM((_C, _D), jnp.float32) for _ in range(_NB)]
    scratch += [pltpu.SemaphoreType.DMA for _ in range(2 * _NB)]

    @functools.partial(
        pl.kernel,
        mesh=mesh,
        out_type=jax.ShapeDtypeStruct((B, _D), jnp.float32),
        compiler_params=pltpu.CompilerParams(use_tc_tiling_on_sc=False),
        scratch_types=scratch,
    )
    def k(tokens_hbm, table_hbm, out_hbm, idx_v, *bufs_and_sems):
        rows = bufs_and_sems[:_NB]
        gsems = bufs_and_sems[_NB : 2 * _NB]
        ssems = bufs_and_sems[2 * _NB : 3 * _NB]

        wid = lax.axis_index("s") * info.num_cores + lax.axis_index("c")
        base = wid * per_w
        pltpu.sync_copy(tokens_hbm.at[wid], idx_v)

        def gather(j, b):
            return pltpu.async_copy(table_hbm.at[idx_v.at[j]], rows[b], gsems[b])

        def scatter_slice(j):
            return out_hbm.at[pl.ds(base + j * _C, _C)]

        for q in range(_LEAD):
            gather(q, q)

        @pl.loop(0, nch // _NB)
        def _outer(o):
            for b in range(_NB):
                j = o * _NB + b
                pltpu.make_async_copy(
                    table_hbm.at[idx_v.at[j]], rows[b], gsems[b]
                ).wait()

                buf = rows[b]

                @plsc.parallel_loop(0, _C, unroll=4)
                def _scale(i):
                    for t in range(_D // 16):
                        sl = pl.ds(t * 16, 16)
                        buf[i, sl] = buf[i, sl] * _SCALE

                pltpu.async_copy(buf, scatter_slice(j), ssems[b])

                jf = j + _LEAD
                bf = (b + _LEAD) % _NB

                @pl.when(jf < nch)
                def _prefetch():
                    @pl.when(jf >= _NB)
                    def _drain():
                        pltpu.make_async_copy(
                            rows[bf], scatter_slice(jf - _NB), ssems[bf]
                        ).wait()

                    gather(jf, bf)

        for b in range(_NB):
            pltpu.make_async_copy(
                rows[b], scatter_slice(nch - _NB + b), ssems[b]
            ).wait()

    return k


def kernel(tokens, table):
    s0, s1 = tokens.shape
    b = s0 * s1
    flat = tokens.reshape(-1).astype(jnp.int32)
    info = plsc.get_sparse_core_info()
    nw = info.num_cores * info.num_subcores
    grain = nw * _C * _NB
    b_pad = (b + grain - 1) // grain * grain
    if b_pad != b:
        flat = jnp.pad(flat, (0, b_pad - b))
    tokens3d = flat.reshape(nw, b_pad // (nw * _C), _C)
    out = _make_kernel(b_pad)(tokens3d, table)
    return out[:b].reshape(s0, s1, _D)
